# Initial kernel scaffold; baseline (speedup 1.0000x reference)
import jax
import jax.numpy as jnp
from jax.experimental import pallas as pl


def kernel(input, adj, W, a):
    raise NotImplementedError("write your pallas kernel here")


# speedup vs baseline: 2545.2677x; 2545.2677x over previous
import jax
import jax.numpy as jnp
from jax.experimental import pallas as pl

_N = 2048
_TILE = 256
_ALPHA = 0.2


def _gat_tile_kernel(x_ref, x_tile_ref, adj_ref, w_ref, a_ref, out_ref):
    f = w_ref.shape[1]
    h_all = jnp.dot(x_ref[...], w_ref[...], preferred_element_type=jnp.float32)
    a_vec = a_ref[...]
    a_src = a_vec[:, :f]
    a_dst = a_vec[:, f:]

    h_i = jnp.dot(x_tile_ref[...], w_ref[...], preferred_element_type=jnp.float32)

    s = jax.lax.dot_general(h_i, a_src, (((1,), (1,)), ((), ())),
                            preferred_element_type=jnp.float32)
    t = jax.lax.dot_general(a_dst, h_all, (((1,), (1,)), ((), ())),
                            preferred_element_type=jnp.float32)

    z = s + t
    lrelu = jnp.where(z >= 0, z, _ALPHA * z)
    e = jnp.exp(-lrelu)
    e = jnp.where(adj_ref[...] != 0, e, 0.0)

    rowsum = jnp.sum(e, axis=1, keepdims=True)
    hp = jnp.dot(e, h_all, preferred_element_type=jnp.float32)
    hp = hp / rowsum
    out_ref[...] = jnp.where(hp > 0, hp, jnp.exp(hp) - 1.0)


def kernel(input, adj, W, a):
    n, in_f = input.shape
    out_f = W.shape[1]
    adj_i8 = adj.astype(jnp.int8)
    grid = (n // _TILE,)
    return pl.pallas_call(
        _gat_tile_kernel,
        grid=grid,
        in_specs=[
            pl.BlockSpec((n, in_f), lambda i: (0, 0)),
            pl.BlockSpec((_TILE, in_f), lambda i: (i, 0)),
            pl.BlockSpec((_TILE, n), lambda i: (i, 0)),
            pl.BlockSpec((in_f, out_f), lambda i: (0, 0)),
            pl.BlockSpec((1, 2 * out_f), lambda i: (0, 0)),
        ],
        out_specs=pl.BlockSpec((_TILE, out_f), lambda i: (i, 0)),
        out_shape=jax.ShapeDtypeStruct((n, out_f), jnp.float32),
    )(input, input, adj_i8, W, a)
